# baseline (speedup 1.0000x reference)
import functools

import jax
import jax.numpy as jnp
from jax import lax
from jax.experimental import pallas as pl
from jax.experimental.pallas import tpu as pltpu
from jax.experimental.pallas import tpu_sc as plsc

VOCAB = 25
EMBED_DIM = 32

_ROWS = 16384
_COLS = 200

_NC = 2
_NS = 16
_NW = _NC * _NS
_IW = _ROWS // _NW

_VPAD = 32

_mesh = plsc.VectorSubcoreMesh(core_axis_name="c", subcore_axis_name="s")


@functools.partial(
    pl.kernel,
    mesh=_mesh,
    out_type=jax.ShapeDtypeStruct((_COLS, EMBED_DIM, _ROWS), jnp.float32),
    scratch_types=[
        pltpu.VMEM((_VPAD, EMBED_DIM), jnp.float32),
        pltpu.VMEM((8, _IW), jnp.int32),
        pltpu.VMEM((EMBED_DIM, _IW), jnp.float32),
        pltpu.VMEM((EMBED_DIM, _IW), jnp.float32),
        pltpu.SemaphoreType.DMA,
        pltpu.SemaphoreType.DMA,
    ],
    compiler_params=pltpu.CompilerParams(
        use_tc_tiling_on_sc=True, needs_layout_passes=False),
)
def _embed_kernel(tpad_hbm, xt_hbm, out_hbm, table_v, xb_v, ob0, ob1, so0, so1):
    wid = lax.axis_index("s") * _NC + lax.axis_index("c")
    ibase = wid * _IW
    ob = (ob0, ob1)
    so = (so0, so1)

    pltpu.sync_copy(tpad_hbm, table_v)

    def outer(g, carry):
        for par in range(2):
            t = g * 2 + par

            @pl.when(t % 8 == 0)
            def _(t=t):
                t8 = pl.multiple_of(t, 8)
                pltpu.sync_copy(
                    xt_hbm.at[pl.ds(t8, 8), pl.ds(ibase, _IW)], xb_v)

            @pl.when(t >= 2)
            def _(par=par):
                pltpu.make_async_copy(
                    ob[par],
                    out_hbm.at[0, :, pl.ds(ibase, _IW)],
                    so[par]).wait()

            row = t % 8

            @plsc.parallel_loop(0, _IW, step=16, unroll=4)
            def fill(m, par=par, row=row):
                idx16 = xb_v[row, pl.ds(m, 16)]
                for k in range(EMBED_DIM):
                    kvec = jnp.full((16,), k, jnp.int32)
                    ob[par][k, pl.ds(m, 16)] = plsc.load_gather(
                        table_v, [kvec, idx16])

            pltpu.async_copy(
                ob[par], out_hbm.at[t, :, pl.ds(ibase, _IW)], so[par])
        return carry

    lax.fori_loop(0, _COLS // 2, outer, 0)

    for par in range(2):
        pltpu.make_async_copy(
            ob[par], out_hbm.at[0, :, pl.ds(ibase, _IW)], so[par]).wait()


def kernel(x, table):
    xt = jnp.transpose(x.astype(jnp.int32))
    tpad = jnp.pad(jnp.transpose(table), ((0, 0), (0, _VPAD - VOCAB)))
    outk = _embed_kernel(tpad, xt)
    return jnp.transpose(outk, (2, 0, 1))


# speedup vs baseline: 1.3366x; 1.3366x over previous
import functools

import jax
import jax.numpy as jnp
from jax import lax
from jax.experimental import pallas as pl
from jax.experimental.pallas import tpu as pltpu
from jax.experimental.pallas import tpu_sc as plsc

VOCAB = 25
EMBED_DIM = 32

_ROWS = 16384
_COLS = 200

_NC = 2
_NS = 16
_NW = _NC * _NS
_IW = _ROWS // _NW

_VPAD = 32

_mesh = plsc.VectorSubcoreMesh(core_axis_name="c", subcore_axis_name="s")


@functools.partial(
    pl.kernel,
    mesh=_mesh,
    out_type=jax.ShapeDtypeStruct((_COLS, EMBED_DIM, _ROWS), jnp.float32),
    scratch_types=[
        pltpu.VMEM((_VPAD, EMBED_DIM), jnp.float32),
        pltpu.VMEM((8, _IW), jnp.int32),
        pltpu.VMEM((EMBED_DIM, _IW), jnp.float32),
        pltpu.VMEM((EMBED_DIM, _IW), jnp.float32),
        pltpu.SemaphoreType.DMA,
        pltpu.SemaphoreType.DMA,
    ],
    compiler_params=pltpu.CompilerParams(
        use_tc_tiling_on_sc=True, needs_layout_passes=False),
)
def _embed_kernel(tpad_hbm, xt_hbm, out_hbm, table_v, xb_v, ob0, ob1, so0, so1):
    wid = lax.axis_index("s") * _NC + lax.axis_index("c")
    ibase = wid * _IW
    ob = (ob0, ob1)
    so = (so0, so1)

    pltpu.sync_copy(tpad_hbm, table_v)

    def outer(g, carry):
        for par in range(2):
            t = g * 2 + par

            @pl.when(t % 8 == 0)
            def _(t=t):
                t8 = pl.multiple_of(t, 8)
                pltpu.sync_copy(
                    xt_hbm.at[pl.ds(t8, 8), pl.ds(ibase, _IW)], xb_v)

            @pl.when(t >= 2)
            def _(par=par):
                pltpu.make_async_copy(
                    ob[par],
                    out_hbm.at[0, :, pl.ds(ibase, _IW)],
                    so[par]).wait()

            row = t % 8

            @plsc.parallel_loop(0, _IW, step=16, unroll=2)
            def fill(m, par=par, row=row):
                idx16 = xb_v[row, pl.ds(m, 16)]
                for k in range(EMBED_DIM):
                    kvec = jnp.full((16,), k, jnp.int32)
                    ob[par][k, pl.ds(m, 16)] = plsc.load_gather(
                        table_v, [kvec, idx16])

            pltpu.async_copy(
                ob[par], out_hbm.at[t, :, pl.ds(ibase, _IW)], so[par])
        return carry

    lax.fori_loop(0, _COLS // 2, outer, 0)

    for par in range(2):
        pltpu.make_async_copy(
            ob[par], out_hbm.at[0, :, pl.ds(ibase, _IW)], so[par]).wait()


def kernel(x, table):
    xt = jnp.transpose(x.astype(jnp.int32))
    tpad = jnp.pad(jnp.transpose(table), ((0, 0), (0, _VPAD - VOCAB)))
    outk = _embed_kernel(tpad, xt)
    return jnp.transpose(outk, (2, 0, 1))
